# Initial kernel scaffold; baseline (speedup 1.0000x reference)
import jax
import jax.numpy as jnp
from jax.experimental import pallas as pl


def kernel(user_ids, item_pos_ids, item_neg_ids, h, r, pos_t, neg_t, user_embed, item_embed, entity_embed, relation_embed, trans_M):
    raise NotImplementedError("write your pallas kernel here")


# speedup vs baseline: 1.4627x; 1.4627x over previous
import functools

import jax
import jax.numpy as jnp
from jax import lax
from jax.experimental import pallas as pl
from jax.experimental.pallas import tpu as pltpu
from jax.experimental.pallas import tpu_sc as plsc

_B = 16384
_D = 32
_RD = 32
_NREL = 64
_NW = 32
_BPW = _B // _NW
_NG = 8
_RG = _NREL // _NG
_KP = _RG * _D
_NP = _NG * _RD
_BLK = 1024
_NB = _B // _BLK
_CF_LAMBDA = 1e-05
_KG_LAMBDA = 1e-05


def _sc_gather_kernel(ue, ie, ee, uid_h, ip_h, in_h, h_h, pt_h, nt_h, out,
                      i_uid, i_ip, i_in, i_h, i_pt, i_nt, rb0, rb1, s0, s1):
    wid = lax.axis_index("s") * 2 + lax.axis_index("c")
    base = wid * _BPW
    sl = pl.ds(base, _BPW)
    pltpu.sync_copy(uid_h.at[sl], i_uid)
    pltpu.sync_copy(ip_h.at[sl], i_ip)
    pltpu.sync_copy(in_h.at[sl], i_in)
    pltpu.sync_copy(h_h.at[sl], i_h)
    pltpu.sync_copy(pt_h.at[sl], i_pt)
    pltpu.sync_copy(nt_h.at[sl], i_nt)
    tasks = [(ee, i_h), (ee, i_pt), (ee, i_nt), (ue, i_uid),
             (ie, i_ip), (ie, i_in), (ee, i_ip), (ee, i_in)]
    bufs = (rb0, rb1)
    sems = (s0, s1)
    copies = [None, None]
    tbl0, idx0 = tasks[0]
    copies[0] = pltpu.async_copy(tbl0.at[idx0], bufs[0], sems[0])
    for g in range(8):
        copies[g % 2].wait()
        if g + 1 < 8:
            tbl, idx = tasks[g + 1]
            copies[(g + 1) % 2] = pltpu.async_copy(
                tbl.at[idx], bufs[(g + 1) % 2], sems[(g + 1) % 2])
        pltpu.sync_copy(bufs[g % 2], out.at[g, sl])


def _sc_gather(user_embed, item_embed, entity_embed, uid, ip, inn, h, pt, nt):
    mesh = plsc.VectorSubcoreMesh(core_axis_name="c", subcore_axis_name="s")
    k = functools.partial(
        pl.kernel,
        mesh=mesh,
        compiler_params=pltpu.CompilerParams(use_tc_tiling_on_sc=False),
        out_type=jax.ShapeDtypeStruct((8, _B, _D), jnp.float32),
        scratch_types=[
            pltpu.VMEM((_BPW,), jnp.int32),
            pltpu.VMEM((_BPW,), jnp.int32),
            pltpu.VMEM((_BPW,), jnp.int32),
            pltpu.VMEM((_BPW,), jnp.int32),
            pltpu.VMEM((_BPW,), jnp.int32),
            pltpu.VMEM((_BPW,), jnp.int32),
            pltpu.VMEM((_BPW, _D), jnp.float32),
            pltpu.VMEM((_BPW, _D), jnp.float32),
            pltpu.SemaphoreType.DMA,
            pltpu.SemaphoreType.DMA,
        ],
    )(_sc_gather_kernel)
    return k(user_embed, item_embed, entity_embed, uid, ip, inn, h, pt, nt)


def _logsig(x):
    return jnp.minimum(x, 0.0) - jnp.log1p(jnp.exp(-jnp.abs(x)))


def _nrm(v):
    n = jnp.sqrt(jnp.sum(v * v, axis=1, keepdims=True))
    return v / jnp.maximum(n, 1e-12)


def _l2h(v):
    return 0.5 * jnp.sum(v * v)


def _tc_body(g_ref, r_ref, w_ref, re_ref, out_ref, acc_ref):
    i = pl.program_id(0)

    @pl.when(i == 0)
    def _():
        for k in range(9):
            acc_ref[k] = 0.0

    r_blk = r_ref[...]
    rl = lax.bitwise_and(r_blk, _RG - 1)
    gg = lax.shift_right_logical(r_blk, 3)

    h_e = g_ref[0]
    pt_e = g_ref[1]
    nt_e = g_ref[2]

    x3 = jnp.concatenate([h_e, pt_e, nt_e], axis=0)
    xt = pltpu.repeat(x3, _RG, axis=1)
    col = lax.broadcasted_iota(jnp.int32, (3 * _BLK, _KP), 1)
    rlcol = lax.shift_right_logical(col, 5)
    rl3 = jnp.concatenate([rl, rl, rl], axis=0)
    a = jnp.where(rlcol == rl3, xt, 0.0)
    rm_big = jnp.dot(a, w_ref[...], preferred_element_type=jnp.float32)
    gg3 = jnp.concatenate([gg, gg, gg], axis=0)
    rm3 = jnp.zeros((3 * _BLK, _RD), jnp.float32)
    for g in range(_NG):
        rm3 = rm3 + jnp.where(gg3 == g, rm_big[:, g * _RD:(g + 1) * _RD], 0.0)
    rmh = rm3[0:_BLK]
    rmpt = rm3[_BLK:2 * _BLK]
    rmnt = rm3[2 * _BLK:]

    oh = (lax.broadcasted_iota(jnp.int32, (_BLK, _NREL), 1) == r_blk)
    r_e = jnp.dot(oh.astype(jnp.float32), re_ref[...],
                  preferred_element_type=jnp.float32)

    rmh_n = _nrm(rmh)
    re_n = _nrm(r_e)
    rmpt_n = _nrm(rmpt)
    rmnt_n = _nrm(rmnt)
    pos_sc = jnp.sum((rmh_n + re_n - rmpt_n) ** 2, axis=1, keepdims=True)
    neg_sc = jnp.sum((rmh_n + re_n - rmnt_n) ** 2, axis=1, keepdims=True)
    kg_ls = jnp.sum(-_logsig(neg_sc - pos_sc))

    u_e = g_ref[3]
    ip_cf = g_ref[4] + g_ref[6]
    in_cf = g_ref[5] + g_ref[7]
    pos_s = jnp.sum(u_e * ip_cf, axis=1, keepdims=True)
    neg_s = jnp.sum(u_e * in_cf, axis=1, keepdims=True)
    cf_ls = jnp.sum(-_logsig(pos_s - neg_s))

    acc_ref[0] = acc_ref[0] + kg_ls
    acc_ref[1] = acc_ref[1] + _l2h(rmh_n)
    acc_ref[2] = acc_ref[2] + _l2h(re_n)
    acc_ref[3] = acc_ref[3] + _l2h(rmpt_n)
    acc_ref[4] = acc_ref[4] + _l2h(rmnt_n)
    acc_ref[5] = acc_ref[5] + cf_ls
    acc_ref[6] = acc_ref[6] + _l2h(u_e)
    acc_ref[7] = acc_ref[7] + _l2h(ip_cf)
    acc_ref[8] = acc_ref[8] + _l2h(in_cf)

    @pl.when(i == _NB - 1)
    def _():
        bf = jnp.float32(_B)
        kg_total = acc_ref[0] / bf + _KG_LAMBDA * (
            (acc_ref[1] + acc_ref[2] + acc_ref[3] + acc_ref[4]) / bf)
        cf_total = acc_ref[5] / bf + _CF_LAMBDA * (
            (acc_ref[6] + acc_ref[7] + acc_ref[8]) / bf)
        out_ref[0, 0] = kg_total + cf_total


def _tc_loss(gath, r2d, w_big, rel_emb):
    return pl.pallas_call(
        _tc_body,
        grid=(_NB,),
        in_specs=[
            pl.BlockSpec((8, _BLK, _D), lambda i: (0, i, 0)),
            pl.BlockSpec((_BLK, 1), lambda i: (i, 0)),
            pl.BlockSpec((_KP, _NP), lambda i: (0, 0)),
            pl.BlockSpec((_NREL, _RD), lambda i: (0, 0)),
        ],
        out_specs=pl.BlockSpec((1, 1), lambda i: (0, 0),
                               memory_space=pltpu.SMEM),
        out_shape=jax.ShapeDtypeStruct((1, 1), jnp.float32),
        scratch_shapes=[pltpu.SMEM((16,), jnp.float32)],
    )(gath, r2d, w_big, rel_emb)


def kernel(user_ids, item_pos_ids, item_neg_ids, h, r, pos_t, neg_t,
           user_embed, item_embed, entity_embed, relation_embed, trans_M):
    uid = user_ids.astype(jnp.int32)
    ip = item_pos_ids.astype(jnp.int32)
    inn = item_neg_ids.astype(jnp.int32)
    h32 = h.astype(jnp.int32)
    pt = pos_t.astype(jnp.int32)
    nt = neg_t.astype(jnp.int32)
    gath = _sc_gather(user_embed, item_embed, entity_embed,
                      uid, ip, inn, h32, pt, nt)
    w_big = trans_M.reshape(_NG, _RG, _D, _RD).transpose(1, 2, 0, 3)
    w_big = w_big.reshape(_KP, _NP)
    r2d = r.astype(jnp.int32).reshape(_B, 1)
    out = _tc_loss(gath, r2d, w_big, relation_embed)
    return out.reshape(())
